# baseline (speedup 1.0000x reference)
import functools

import jax
import jax.numpy as jnp
from jax import lax
from jax.experimental import pallas as pl
from jax.experimental.pallas import tpu as pltpu
from jax.experimental.pallas import tpu_sc as plsc

N = 10000
E = 160000
D_IN = 256
D_H = 1024
L = 4

NC = 2
NS = 16
K = 128

E2 = E + N
P = ((E2 + NC * NS * K - 1) // (NC * NS * K)) * (NC * NS * K)
PT = P // NS
NB = PT // K

NPAD = 10240
RPT = NPAD // NS
TRASH = NPAD
ACC_R = NPAD + 16
NF = D_H // 128
FPC = NF // NC

RB = 1000
GRID = N // RB


_PT32 = P // (NC * NS)
_NB32 = _PT32 // K
DW = 16


def _deg_body(de_hbm, ones_hbm, zer_hbm, degp_hbm,
              acc, dst_v, ones_v, _sem):
    c = lax.axis_index("c")
    s = lax.axis_index("s")
    pltpu.sync_copy(ones_hbm, ones_v)
    pltpu.sync_copy(zer_hbm, acc.at[pl.ds(s * RPT, RPT)])
    pltpu.sync_copy(zer_hbm.at[pl.ds(0, 16)], acc.at[pl.ds(NPAD, 16)])
    plsc.subcore_barrier()

    base = (c * NS + s) * _PT32

    def batch(i, _):
        pltpu.sync_copy(de_hbm.at[pl.ds(base + i * K, K)], dst_v)
        pltpu.sync_copy(ones_v, acc.at[dst_v], add=True)
        return 0
    lax.fori_loop(0, _NB32, batch, 0)
    plsc.subcore_barrier()

    pltpu.sync_copy(acc.at[pl.ds(s * RPT, RPT)],
                    degp_hbm.at[c, pl.ds(s * RPT, RPT)])


@functools.cache
def _build_deg_kernel():
    return functools.partial(
        pl.kernel,
        out_type=jax.ShapeDtypeStruct((NC, NPAD, DW), jnp.float32),
        mesh=plsc.VectorSubcoreMesh(core_axis_name="c", subcore_axis_name="s",
                                    num_cores=NC, num_subcores=NS),
        scratch_types=[
            pltpu.VMEM_SHARED((ACC_R, DW), jnp.float32),
            pltpu.VMEM((K,), jnp.int32),
            pltpu.VMEM((K, DW), jnp.float32),
            pltpu.SemaphoreType.DMA,
        ],
    )(_deg_body)


def _deg_kernel(de):
    ones = jnp.ones((K, DW), jnp.float32)
    zer = jnp.zeros((RPT, DW), jnp.float32)
    return _build_deg_kernel()(de, ones, zer)


def _agg_one_chunk(s, de_hbm, zrows_hbm, xs_hbm, out_hbm,
                   acc, se_v, dst_a, dst_b, rows_a, rows_b,
                   gsem_a, gsem_b, ssem_a, ssem_b, isem_a, isem_b):
    pltpu.sync_copy(zrows_hbm, acc.at[pl.ds(s * RPT, RPT)])
    plsc.subcore_barrier()

    base = s * PT

    def load_dst(i, dst_v, sem):
        pltpu.async_copy(de_hbm.at[pl.ds(base + i * K, K)], dst_v, sem)

    def start_gather(i, buf, sem):
        pltpu.async_copy(xs_hbm.at[se_v.at[pl.ds(i * K, K)]], buf, sem)

    def wait_gather(buf, sem):
        pltpu.make_async_copy(xs_hbm.at[se_v.at[pl.ds(0, K)]], buf, sem).wait()

    def wait_dst(dst_v, sem):
        pltpu.make_async_copy(de_hbm.at[pl.ds(base, K)], dst_v, sem).wait()

    def wait_scatter(buf, dst_v, sem):
        pltpu.make_async_copy(buf, acc.at[dst_v], sem).wait()

    load_dst(0, dst_a, isem_a)
    start_gather(0, rows_a, gsem_a)

    def step(g, _):
        i0 = 2 * g
        i1 = i0 + 1

        @pl.when(g > 0)
        def _():
            wait_scatter(rows_b, dst_b, ssem_b)
        load_dst(i1, dst_b, isem_b)
        start_gather(i1, rows_b, gsem_b)
        wait_gather(rows_a, gsem_a)
        wait_dst(dst_a, isem_a)
        pltpu.async_copy(rows_a, acc.at[dst_a], ssem_a, add=True)

        @pl.when(i1 + 1 < NB)
        def _():
            wait_scatter(rows_a, dst_a, ssem_a)
            load_dst(i1 + 1, dst_a, isem_a)
            start_gather(i1 + 1, rows_a, gsem_a)
        wait_gather(rows_b, gsem_b)
        wait_dst(dst_b, isem_b)
        pltpu.async_copy(rows_b, acc.at[dst_b], ssem_b, add=True)
        return 0
    lax.fori_loop(0, NB // 2, step, 0)
    wait_scatter(rows_a, dst_a, ssem_a)
    wait_scatter(rows_b, dst_b, ssem_b)
    plsc.subcore_barrier()

    @pl.when(s < NS - 1)
    def _():
        pltpu.sync_copy(acc.at[pl.ds(s * RPT, RPT)],
                        out_hbm.at[pl.ds(s * RPT, RPT)])

    @pl.when(s == NS - 1)
    def _():
        pltpu.sync_copy(acc.at[pl.ds((NS - 1) * RPT, N - (NS - 1) * RPT)],
                        out_hbm.at[pl.ds((NS - 1) * RPT, N - (NS - 1) * RPT)])

    plsc.subcore_barrier()


def _agg_body(se_hbm, de_hbm, zrows_hbm, *rest):
    xs_refs = rest[:NF]
    out_refs = rest[NF:2 * NF]
    (acc, se_v, dst_a, dst_b, rows_a, rows_b,
     gsem_a, gsem_b, ssem_a, ssem_b, isem_a, isem_b) = rest[2 * NF:]
    c = lax.axis_index("c")
    s = lax.axis_index("s")
    pltpu.sync_copy(se_hbm.at[pl.ds(s * PT, PT)], se_v)
    for j in range(FPC):
        @pl.when(c == 0)
        def _(j=j):
            _agg_one_chunk(s, de_hbm, zrows_hbm,
                           xs_refs[j], out_refs[j],
                           acc, se_v, dst_a, dst_b, rows_a, rows_b,
                           gsem_a, gsem_b, ssem_a, ssem_b, isem_a, isem_b)

        @pl.when(c == 1)
        def _(j=j):
            _agg_one_chunk(s, de_hbm, zrows_hbm,
                           xs_refs[FPC + j], out_refs[FPC + j],
                           acc, se_v, dst_a, dst_b, rows_a, rows_b,
                           gsem_a, gsem_b, ssem_a, ssem_b, isem_a, isem_b)


@functools.cache
def _build_agg_kernel():
    return functools.partial(
        pl.kernel,
        out_type=[jax.ShapeDtypeStruct((N, 128), jnp.float32)] * NF,
        mesh=plsc.VectorSubcoreMesh(core_axis_name="c", subcore_axis_name="s",
                                    num_cores=NC, num_subcores=NS),
        scratch_types=[
            pltpu.VMEM_SHARED((ACC_R, 128), jnp.float32),
            pltpu.VMEM((PT,), jnp.int32),
            pltpu.VMEM((K,), jnp.int32),
            pltpu.VMEM((K,), jnp.int32),
            pltpu.VMEM((K, 128), jnp.float32),
            pltpu.VMEM((K, 128), jnp.float32),
            pltpu.SemaphoreType.DMA,
            pltpu.SemaphoreType.DMA,
            pltpu.SemaphoreType.DMA,
            pltpu.SemaphoreType.DMA,
            pltpu.SemaphoreType.DMA,
            pltpu.SemaphoreType.DMA,
        ],
    )(_agg_body)


def _agg_kernel(se, de, zrows, *xs_list):
    return _build_agg_kernel()(se, de, zrows, *xs_list)


_PREC = lax.Precision.DEFAULT


def _tc1_body(x_ref, w_ref, degt_ref, *outs):
    xs_outs = outs[:NF]
    dinv_ref = outs[NF]
    deg = jnp.sum(degt_ref[...], axis=1, keepdims=True)
    dinv = lax.rsqrt(deg)
    xw = jnp.dot(x_ref[...], w_ref[...],
                 preferred_element_type=jnp.float32, precision=_PREC)
    xs = xw * dinv
    for f in range(NF):
        xs_outs[f][...] = xs[:, f * 128:(f + 1) * 128]
    dinv_ref[...] = dinv


def _tc1(x, w1, degt):
    return pl.pallas_call(
        _tc1_body,
        grid=(GRID,),
        in_specs=[
            pl.BlockSpec((RB, D_IN), lambda i: (i, 0)),
            pl.BlockSpec((D_IN, D_H), lambda i: (0, 0)),
            pl.BlockSpec((RB, NC), lambda i: (i, 0)),
        ],
        out_specs=[pl.BlockSpec((RB, 128), lambda i: (i, 0))] * NF
                  + [pl.BlockSpec((RB, 1), lambda i: (i, 0))],
        out_shape=[jax.ShapeDtypeStruct((N, 128), jnp.float32)] * NF
                  + [jax.ShapeDtypeStruct((N, 1), jnp.float32)],
    )(x, w1, degt)


def _ln_relu(a, g, b):
    mu = jnp.mean(a, axis=-1, keepdims=True)
    var = jnp.mean((a - mu) ** 2, axis=-1, keepdims=True)
    h = g * (a - mu) / jnp.sqrt(var + 1e-5) + b
    return jnp.maximum(h, 0.0)


def _tcmid_body(*refs):
    s_refs = refs[:NF]
    dinv_ref, b_ref, g_ref, beta_ref, w_ref = refs[NF:NF + 5]
    xs_outs = refs[NF + 5:]
    sb = jnp.concatenate([r[...] for r in s_refs], axis=1)
    dinv = dinv_ref[...]
    a = sb * dinv + b_ref[...]
    h = _ln_relu(a, g_ref[...], beta_ref[...])
    xw = jnp.dot(h, w_ref[...],
                 preferred_element_type=jnp.float32, precision=_PREC)
    xs = xw * dinv
    for f in range(NF):
        xs_outs[f][...] = xs[:, f * 128:(f + 1) * 128]


def _tcmid(s_list, dinv, b, g, beta, w):
    return pl.pallas_call(
        _tcmid_body,
        grid=(GRID,),
        in_specs=[pl.BlockSpec((RB, 128), lambda i: (i, 0))] * NF + [
            pl.BlockSpec((RB, 1), lambda i: (i, 0)),
            pl.BlockSpec((1, D_H), lambda i: (0, 0)),
            pl.BlockSpec((1, D_H), lambda i: (0, 0)),
            pl.BlockSpec((1, D_H), lambda i: (0, 0)),
            pl.BlockSpec((D_H, D_H), lambda i: (0, 0)),
        ],
        out_specs=[pl.BlockSpec((RB, 128), lambda i: (i, 0))] * NF,
        out_shape=[jax.ShapeDtypeStruct((N, 128), jnp.float32)] * NF,
    )(*s_list, dinv, b, g, beta, w)


def _tcfin_body(*refs):
    s_refs = refs[:NF]
    dinv_ref, b_ref, g_ref, beta_ref = refs[NF:NF + 4]
    out_ref = refs[NF + 4]
    i = pl.program_id(0)
    sb = jnp.concatenate([r[...] for r in s_refs], axis=1)
    a = sb * dinv_ref[...] + b_ref[...]
    h = _ln_relu(a, g_ref[...], beta_ref[...])
    part = jnp.sum(h, axis=0, keepdims=True)

    @pl.when(i == 0)
    def _():
        out_ref[...] = part

    @pl.when(i > 0)
    def _():
        out_ref[...] = out_ref[...] + part

    @pl.when(i == GRID - 1)
    def _():
        out_ref[...] = out_ref[...] * (1.0 / N)


def _tcfin(s_list, dinv, b, g, beta):
    return pl.pallas_call(
        _tcfin_body,
        grid=(GRID,),
        in_specs=[pl.BlockSpec((RB, 128), lambda i: (i, 0))] * NF + [
            pl.BlockSpec((RB, 1), lambda i: (i, 0)),
            pl.BlockSpec((1, D_H), lambda i: (0, 0)),
            pl.BlockSpec((1, D_H), lambda i: (0, 0)),
            pl.BlockSpec((1, D_H), lambda i: (0, 0)),
        ],
        out_specs=pl.BlockSpec((1, D_H), lambda i: (0, 0)),
        out_shape=jax.ShapeDtypeStruct((1, D_H), jnp.float32),
    )(*s_list, dinv, b, g, beta)


def kernel(x, edge_index, Ws, bs, gammas, betas):
    src = edge_index[0].astype(jnp.int32)
    dst = edge_index[1].astype(jnp.int32)
    loop = jnp.arange(N, dtype=jnp.int32)
    npad = P - E2
    allsrc = jnp.concatenate([src, loop])
    alldst = jnp.concatenate([dst, loop])
    order = jnp.argsort(allsrc)
    se = jnp.concatenate([allsrc[order], jnp.zeros((npad,), jnp.int32)])
    de = jnp.concatenate([alldst[order], jnp.full((npad,), TRASH, jnp.int32)])

    degp = _deg_kernel(de)
    degt = degp[:, :N, 0].T

    zrows = jnp.zeros((RPT, 128), jnp.float32)

    outs = _tc1(x, Ws[0], degt)
    xs_list, dinv = list(outs[:NF]), outs[NF]
    for i in range(L - 1):
        s_list = list(_agg_kernel(se, de, zrows, *xs_list))
        xs_list = list(_tcmid(s_list, dinv, bs[i].reshape(1, D_H),
                              gammas[i].reshape(1, D_H),
                              betas[i].reshape(1, D_H), Ws[i + 1]))
    s_list = list(_agg_kernel(se, de, zrows, *xs_list))
    return _tcfin(s_list, dinv, bs[L - 1].reshape(1, D_H),
                  gammas[L - 1].reshape(1, D_H),
                  betas[L - 1].reshape(1, D_H))


# speedup vs baseline: 1.2401x; 1.2401x over previous
import functools

import jax
import jax.numpy as jnp
from jax import lax
from jax.experimental import pallas as pl
from jax.experimental.pallas import tpu as pltpu
from jax.experimental.pallas import tpu_sc as plsc

N = 10000
E = 160000
D_IN = 256
D_H = 1024
L = 4

NC = 2
NS = 16
K = 64

E2 = E + N
P = ((E2 + NC * NS * K - 1) // (NC * NS * K)) * (NC * NS * K)
PT = P // NS
NB = PT // K

NPAD = 10240
RPT = NPAD // NS
TRASH = NPAD
ACC_R = NPAD + 16
NF = D_H // 128
FPC = NF // NC

RB = 1000
GRID = N // RB


_PT32 = P // (NC * NS)
_NB32 = _PT32 // K
DW = 16


def _deg_body(de_hbm, ones_hbm, zer_hbm, degp_hbm,
              acc, dst_v, ones_v, _sem):
    c = lax.axis_index("c")
    s = lax.axis_index("s")
    pltpu.sync_copy(ones_hbm, ones_v)
    pltpu.sync_copy(zer_hbm, acc.at[pl.ds(s * RPT, RPT)])
    pltpu.sync_copy(zer_hbm.at[pl.ds(0, 16)], acc.at[pl.ds(NPAD, 16)])
    plsc.subcore_barrier()

    base = (c * NS + s) * _PT32

    def batch(i, _):
        pltpu.sync_copy(de_hbm.at[pl.ds(base + i * K, K)], dst_v)
        pltpu.sync_copy(ones_v, acc.at[dst_v], add=True)
        return 0
    lax.fori_loop(0, _NB32, batch, 0)
    plsc.subcore_barrier()

    pltpu.sync_copy(acc.at[pl.ds(s * RPT, RPT)],
                    degp_hbm.at[c, pl.ds(s * RPT, RPT)])


@functools.cache
def _build_deg_kernel():
    return functools.partial(
        pl.kernel,
        out_type=jax.ShapeDtypeStruct((NC, NPAD, DW), jnp.float32),
        mesh=plsc.VectorSubcoreMesh(core_axis_name="c", subcore_axis_name="s",
                                    num_cores=NC, num_subcores=NS),
        scratch_types=[
            pltpu.VMEM_SHARED((ACC_R, DW), jnp.float32),
            pltpu.VMEM((K,), jnp.int32),
            pltpu.VMEM((K, DW), jnp.float32),
            pltpu.SemaphoreType.DMA,
        ],
    )(_deg_body)


def _deg_kernel(de):
    ones = jnp.ones((K, DW), jnp.float32)
    zer = jnp.zeros((RPT, DW), jnp.float32)
    return _build_deg_kernel()(de, ones, zer)


NSLOT = 4


def _agg_one_chunk(s, de_hbm, zrows_hbm, xs_hbm, out_hbm,
                   acc, se_v, dstv, rows, gsem, ssem, isem):
    pltpu.sync_copy(zrows_hbm, acc.at[pl.ds(s * RPT, RPT)])
    plsc.subcore_barrier()

    base = s * PT

    def load_dst(i, q):
        pltpu.async_copy(de_hbm.at[pl.ds(base + i * K, K)], dstv[q], isem[q])

    def start_gather(i, q):
        pltpu.async_copy(xs_hbm.at[se_v.at[pl.ds(i * K, K)]], rows[q], gsem[q])

    def wait_gather(q):
        pltpu.make_async_copy(xs_hbm.at[se_v.at[pl.ds(0, K)]],
                              rows[q], gsem[q]).wait()

    def wait_dst(q):
        pltpu.make_async_copy(de_hbm.at[pl.ds(base, K)],
                              dstv[q], isem[q]).wait()

    def wait_scatter(q):
        pltpu.make_async_copy(rows[q], acc.at[dstv[q]], ssem[q]).wait()

    load_dst(0, 0)
    start_gather(0, 0)
    load_dst(1, 1)
    start_gather(1, 1)

    def step(m, _):
        for q in range(NSLOT):
            i = NSLOT * m + q
            wait_gather(q)
            wait_dst(q)
            pltpu.async_copy(rows[q], acc.at[dstv[q]], ssem[q], add=True)
            r = (q + 2) % NSLOT

            @pl.when(i + 2 < NB)
            def _(i=i, r=r):
                @pl.when(i >= 2)
                def _():
                    wait_scatter(r)
                load_dst(i + 2, r)
                start_gather(i + 2, r)
        return 0
    lax.fori_loop(0, NB // NSLOT, step, 0)
    for q in range(NSLOT):
        wait_scatter(q)
    plsc.subcore_barrier()

    @pl.when(s < NS - 1)
    def _():
        pltpu.sync_copy(acc.at[pl.ds(s * RPT, RPT)],
                        out_hbm.at[pl.ds(s * RPT, RPT)])

    @pl.when(s == NS - 1)
    def _():
        pltpu.sync_copy(acc.at[pl.ds((NS - 1) * RPT, N - (NS - 1) * RPT)],
                        out_hbm.at[pl.ds((NS - 1) * RPT, N - (NS - 1) * RPT)])

    plsc.subcore_barrier()


def _agg_body(se_hbm, de_hbm, zrows_hbm, *rest):
    xs_refs = rest[:NF]
    out_refs = rest[NF:2 * NF]
    rest = rest[2 * NF:]
    acc, se_v = rest[0], rest[1]
    dstv = rest[2:2 + NSLOT]
    rows = rest[2 + NSLOT:2 + 2 * NSLOT]
    gsem = rest[2 + 2 * NSLOT:2 + 3 * NSLOT]
    ssem = rest[2 + 3 * NSLOT:2 + 4 * NSLOT]
    isem = rest[2 + 4 * NSLOT:2 + 5 * NSLOT]
    c = lax.axis_index("c")
    s = lax.axis_index("s")
    pltpu.sync_copy(se_hbm.at[pl.ds(s * PT, PT)], se_v)
    for j in range(FPC):
        @pl.when(c == 0)
        def _(j=j):
            _agg_one_chunk(s, de_hbm, zrows_hbm,
                           xs_refs[j], out_refs[j],
                           acc, se_v, dstv, rows, gsem, ssem, isem)

        @pl.when(c == 1)
        def _(j=j):
            _agg_one_chunk(s, de_hbm, zrows_hbm,
                           xs_refs[FPC + j], out_refs[FPC + j],
                           acc, se_v, dstv, rows, gsem, ssem, isem)


@functools.cache
def _build_agg_kernel():
    return functools.partial(
        pl.kernel,
        out_type=[jax.ShapeDtypeStruct((N, 128), jnp.float32)] * NF,
        mesh=plsc.VectorSubcoreMesh(core_axis_name="c", subcore_axis_name="s",
                                    num_cores=NC, num_subcores=NS),
        scratch_types=(
            [pltpu.VMEM_SHARED((ACC_R, 128), jnp.float32),
             pltpu.VMEM((PT,), jnp.int32)]
            + [pltpu.VMEM((K,), jnp.int32)] * NSLOT
            + [pltpu.VMEM((K, 128), jnp.float32)] * NSLOT
            + [pltpu.SemaphoreType.DMA] * (3 * NSLOT)
        ),
    )(_agg_body)


def _agg_kernel(se, de, zrows, *xs_list):
    return _build_agg_kernel()(se, de, zrows, *xs_list)


_PREC = lax.Precision.DEFAULT


def _tc1_body(x_ref, w_ref, degt_ref, *outs):
    xs_outs = outs[:NF]
    dinv_ref = outs[NF]
    deg = jnp.sum(degt_ref[...], axis=1, keepdims=True)
    dinv = lax.rsqrt(deg)
    xw = jnp.dot(x_ref[...], w_ref[...],
                 preferred_element_type=jnp.float32, precision=_PREC)
    xs = xw * dinv
    for f in range(NF):
        xs_outs[f][...] = xs[:, f * 128:(f + 1) * 128]
    dinv_ref[...] = dinv


def _tc1(x, w1, degt):
    return pl.pallas_call(
        _tc1_body,
        grid=(GRID,),
        in_specs=[
            pl.BlockSpec((RB, D_IN), lambda i: (i, 0)),
            pl.BlockSpec((D_IN, D_H), lambda i: (0, 0)),
            pl.BlockSpec((RB, NC), lambda i: (i, 0)),
        ],
        out_specs=[pl.BlockSpec((RB, 128), lambda i: (i, 0))] * NF
                  + [pl.BlockSpec((RB, 1), lambda i: (i, 0))],
        out_shape=[jax.ShapeDtypeStruct((N, 128), jnp.float32)] * NF
                  + [jax.ShapeDtypeStruct((N, 1), jnp.float32)],
    )(x, w1, degt)


def _ln_relu(a, g, b):
    mu = jnp.mean(a, axis=-1, keepdims=True)
    var = jnp.mean((a - mu) ** 2, axis=-1, keepdims=True)
    h = g * (a - mu) / jnp.sqrt(var + 1e-5) + b
    return jnp.maximum(h, 0.0)


def _tcmid_body(*refs):
    s_refs = refs[:NF]
    dinv_ref, b_ref, g_ref, beta_ref, w_ref = refs[NF:NF + 5]
    xs_outs = refs[NF + 5:]
    sb = jnp.concatenate([r[...] for r in s_refs], axis=1)
    dinv = dinv_ref[...]
    a = sb * dinv + b_ref[...]
    h = _ln_relu(a, g_ref[...], beta_ref[...])
    xw = jnp.dot(h, w_ref[...],
                 preferred_element_type=jnp.float32, precision=_PREC)
    xs = xw * dinv
    for f in range(NF):
        xs_outs[f][...] = xs[:, f * 128:(f + 1) * 128]


def _tcmid(s_list, dinv, b, g, beta, w):
    return pl.pallas_call(
        _tcmid_body,
        grid=(GRID,),
        in_specs=[pl.BlockSpec((RB, 128), lambda i: (i, 0))] * NF + [
            pl.BlockSpec((RB, 1), lambda i: (i, 0)),
            pl.BlockSpec((1, D_H), lambda i: (0, 0)),
            pl.BlockSpec((1, D_H), lambda i: (0, 0)),
            pl.BlockSpec((1, D_H), lambda i: (0, 0)),
            pl.BlockSpec((D_H, D_H), lambda i: (0, 0)),
        ],
        out_specs=[pl.BlockSpec((RB, 128), lambda i: (i, 0))] * NF,
        out_shape=[jax.ShapeDtypeStruct((N, 128), jnp.float32)] * NF,
    )(*s_list, dinv, b, g, beta, w)


def _tcfin_body(*refs):
    s_refs = refs[:NF]
    dinv_ref, b_ref, g_ref, beta_ref = refs[NF:NF + 4]
    out_ref = refs[NF + 4]
    i = pl.program_id(0)
    sb = jnp.concatenate([r[...] for r in s_refs], axis=1)
    a = sb * dinv_ref[...] + b_ref[...]
    h = _ln_relu(a, g_ref[...], beta_ref[...])
    part = jnp.sum(h, axis=0, keepdims=True)

    @pl.when(i == 0)
    def _():
        out_ref[...] = part

    @pl.when(i > 0)
    def _():
        out_ref[...] = out_ref[...] + part

    @pl.when(i == GRID - 1)
    def _():
        out_ref[...] = out_ref[...] * (1.0 / N)


def _tcfin(s_list, dinv, b, g, beta):
    return pl.pallas_call(
        _tcfin_body,
        grid=(GRID,),
        in_specs=[pl.BlockSpec((RB, 128), lambda i: (i, 0))] * NF + [
            pl.BlockSpec((RB, 1), lambda i: (i, 0)),
            pl.BlockSpec((1, D_H), lambda i: (0, 0)),
            pl.BlockSpec((1, D_H), lambda i: (0, 0)),
            pl.BlockSpec((1, D_H), lambda i: (0, 0)),
        ],
        out_specs=pl.BlockSpec((1, D_H), lambda i: (0, 0)),
        out_shape=jax.ShapeDtypeStruct((1, D_H), jnp.float32),
    )(*s_list, dinv, b, g, beta)


def kernel(x, edge_index, Ws, bs, gammas, betas):
    src = edge_index[0].astype(jnp.int32)
    dst = edge_index[1].astype(jnp.int32)
    loop = jnp.arange(N, dtype=jnp.int32)
    npad = P - E2
    se = jnp.concatenate([src, loop, jnp.zeros((npad,), jnp.int32)])
    de = jnp.concatenate([dst, loop, jnp.full((npad,), TRASH, jnp.int32)])

    degp = _deg_kernel(de)
    degt = degp[:, :N, 0].T

    zrows = jnp.zeros((RPT, 128), jnp.float32)

    outs = _tc1(x, Ws[0], degt)
    xs_list, dinv = list(outs[:NF]), outs[NF]
    for i in range(L - 1):
        s_list = list(_agg_kernel(se, de, zrows, *xs_list))
        xs_list = list(_tcmid(s_list, dinv, bs[i].reshape(1, D_H),
                              gammas[i].reshape(1, D_H),
                              betas[i].reshape(1, D_H), Ws[i + 1]))
    s_list = list(_agg_kernel(se, de, zrows, *xs_list))
    return _tcfin(s_list, dinv, bs[L - 1].reshape(1, D_H),
                  gammas[L - 1].reshape(1, D_H),
                  betas[L - 1].reshape(1, D_H))


# speedup vs baseline: 1.2575x; 1.0140x over previous
import functools

import jax
import jax.numpy as jnp
from jax import lax
from jax.experimental import pallas as pl
from jax.experimental.pallas import tpu as pltpu
from jax.experimental.pallas import tpu_sc as plsc

N = 10000
E = 160000
D_IN = 256
D_H = 1024
L = 4

NC = 2
NS = 16
K = 128

E2 = E + N
P = ((E2 + NC * NS * K - 1) // (NC * NS * K)) * (NC * NS * K)
PT = P // NS
NB = PT // K

RPT = 632
ACC_R = NS * RPT
TRASH = N
NTR = ACC_R - N
NF = D_H // 128
FPC = NF // NC
WB15 = N - (NS - 1) * RPT

RB = 1000
GRID = N // RB


_PT32 = P // (NC * NS)
_NB32 = _PT32 // K
DW = 16


def _deg_body(de_hbm, ones_hbm, zer_hbm, degp_hbm,
              acc, dst_v, ones_v, _sem):
    c = lax.axis_index("c")
    s = lax.axis_index("s")
    pltpu.sync_copy(ones_hbm, ones_v)
    pltpu.sync_copy(zer_hbm, acc.at[pl.ds(s * RPT, RPT)])
    plsc.subcore_barrier()

    base = (c * NS + s) * _PT32

    def batch(i, _):
        pltpu.sync_copy(de_hbm.at[pl.ds(base + i * K, K)], dst_v)
        pltpu.sync_copy(ones_v, acc.at[dst_v], add=True)
        return 0
    lax.fori_loop(0, _NB32, batch, 0)
    plsc.subcore_barrier()

    pltpu.sync_copy(acc.at[pl.ds(s * RPT, RPT)],
                    degp_hbm.at[c, pl.ds(s * RPT, RPT)])


@functools.cache
def _build_deg_kernel():
    return functools.partial(
        pl.kernel,
        out_type=jax.ShapeDtypeStruct((NC, ACC_R, DW), jnp.float32),
        mesh=plsc.VectorSubcoreMesh(core_axis_name="c", subcore_axis_name="s",
                                    num_cores=NC, num_subcores=NS),
        scratch_types=[
            pltpu.VMEM_SHARED((ACC_R, DW), jnp.float32),
            pltpu.VMEM((K,), jnp.int32),
            pltpu.VMEM((K, DW), jnp.float32),
            pltpu.SemaphoreType.DMA,
        ],
    )(_deg_body)


def _deg_kernel(de):
    ones = jnp.ones((K, DW), jnp.float32)
    zer = jnp.zeros((RPT, DW), jnp.float32)
    return _build_deg_kernel()(de, ones, zer)


NSLOT = 3


def _agg_one_chunk(s, se_hbm, de_hbm, zrows_hbm, xs_hbm, out_hbm,
                   acc, srcv, dstv, rows, gsem, ssem, xsem, dsem):
    pltpu.sync_copy(zrows_hbm, acc.at[pl.ds(s * RPT, RPT)])
    plsc.subcore_barrier()

    base = s * PT

    def load_idx(i, q):
        pltpu.async_copy(se_hbm.at[pl.ds(base + i * K, K)], srcv[q], xsem[q])
        pltpu.async_copy(de_hbm.at[pl.ds(base + i * K, K)], dstv[q], dsem[q])

    def wait_src(q):
        pltpu.make_async_copy(se_hbm.at[pl.ds(base, K)],
                              srcv[q], xsem[q]).wait()

    def wait_dst(q):
        pltpu.make_async_copy(de_hbm.at[pl.ds(base, K)],
                              dstv[q], dsem[q]).wait()

    def start_gather(q):
        pltpu.async_copy(xs_hbm.at[srcv[q]], rows[q], gsem[q])

    def wait_gather(q):
        pltpu.make_async_copy(xs_hbm.at[srcv[q]], rows[q], gsem[q]).wait()

    def wait_scatter(q):
        pltpu.make_async_copy(rows[q], acc.at[dstv[q]], ssem[q]).wait()

    load_idx(0, 0)
    load_idx(1, 1)
    wait_src(0)
    start_gather(0)

    def step(m, _):
        for q in range(NSLOT):
            i0 = NSLOT * m + q
            p = (q + 2) % NSLOT
            n = (q + 1) % NSLOT
            wait_gather(q)
            wait_dst(q)
            pltpu.async_copy(rows[q], acc.at[dstv[q]], ssem[q], add=True)

            @pl.when(i0 >= 1)
            def _(p=p):
                wait_scatter(p)

            @pl.when(i0 + 2 < NB)
            def _(i0=i0, p=p):
                load_idx(i0 + 2, p)

            @pl.when(i0 + 1 < NB)
            def _(n=n):
                wait_src(n)
                start_gather(n)
        return 0
    lax.fori_loop(0, NB // NSLOT, step, 0)
    wait_scatter((NB - 1) % NSLOT)
    plsc.subcore_barrier()

    @pl.when(s < NS - 1)
    def _():
        pltpu.sync_copy(acc.at[pl.ds(s * RPT, RPT)],
                        out_hbm.at[pl.ds(s * RPT, RPT)])

    @pl.when(s == NS - 1)
    def _():
        pltpu.sync_copy(acc.at[pl.ds((NS - 1) * RPT, WB15)],
                        out_hbm.at[pl.ds((NS - 1) * RPT, WB15)])

    plsc.subcore_barrier()


def _agg_body(se_hbm, de_hbm, zrows_hbm, *rest):
    xs_refs = rest[:NF]
    out_refs = rest[NF:2 * NF]
    rest = rest[2 * NF:]
    acc = rest[0]
    srcv = rest[1:1 + NSLOT]
    dstv = rest[1 + NSLOT:1 + 2 * NSLOT]
    rows = rest[1 + 2 * NSLOT:1 + 3 * NSLOT]
    gsem = rest[1 + 3 * NSLOT:1 + 4 * NSLOT]
    ssem = rest[1 + 4 * NSLOT:1 + 5 * NSLOT]
    xsem = rest[1 + 5 * NSLOT:1 + 6 * NSLOT]
    dsem = rest[1 + 6 * NSLOT:1 + 7 * NSLOT]
    c = lax.axis_index("c")
    s = lax.axis_index("s")
    for j in range(FPC):
        @pl.when(c == 0)
        def _(j=j):
            _agg_one_chunk(s, se_hbm, de_hbm, zrows_hbm,
                           xs_refs[j], out_refs[j],
                           acc, srcv, dstv, rows, gsem, ssem, xsem, dsem)

        @pl.when(c == 1)
        def _(j=j):
            _agg_one_chunk(s, se_hbm, de_hbm, zrows_hbm,
                           xs_refs[FPC + j], out_refs[FPC + j],
                           acc, srcv, dstv, rows, gsem, ssem, xsem, dsem)


@functools.cache
def _build_agg_kernel():
    return functools.partial(
        pl.kernel,
        out_type=[jax.ShapeDtypeStruct((N, 128), jnp.float32)] * NF,
        mesh=plsc.VectorSubcoreMesh(core_axis_name="c", subcore_axis_name="s",
                                    num_cores=NC, num_subcores=NS),
        scratch_types=(
            [pltpu.VMEM_SHARED((ACC_R, 128), jnp.float32)]
            + [pltpu.VMEM((K,), jnp.int32)] * (2 * NSLOT)
            + [pltpu.VMEM((K, 128), jnp.float32)] * NSLOT
            + [pltpu.SemaphoreType.DMA] * (4 * NSLOT)
        ),
    )(_agg_body)


def _agg_kernel(se, de, zrows, *xs_list):
    return _build_agg_kernel()(se, de, zrows, *xs_list)


_PREC = lax.Precision.DEFAULT


def _tc1_body(x_ref, w_ref, degt_ref, *outs):
    xs_outs = outs[:NF]
    dinv_ref = outs[NF]
    deg = jnp.sum(degt_ref[...], axis=1, keepdims=True)
    dinv = lax.rsqrt(deg)
    xw = jnp.dot(x_ref[...], w_ref[...],
                 preferred_element_type=jnp.float32, precision=_PREC)
    xs = xw * dinv
    for f in range(NF):
        xs_outs[f][...] = xs[:, f * 128:(f + 1) * 128]
    dinv_ref[...] = dinv


def _tc1(x, w1, degt):
    return pl.pallas_call(
        _tc1_body,
        grid=(GRID,),
        in_specs=[
            pl.BlockSpec((RB, D_IN), lambda i: (i, 0)),
            pl.BlockSpec((D_IN, D_H), lambda i: (0, 0)),
            pl.BlockSpec((RB, NC), lambda i: (i, 0)),
        ],
        out_specs=[pl.BlockSpec((RB, 128), lambda i: (i, 0))] * NF
                  + [pl.BlockSpec((RB, 1), lambda i: (i, 0))],
        out_shape=[jax.ShapeDtypeStruct((N, 128), jnp.float32)] * NF
                  + [jax.ShapeDtypeStruct((N, 1), jnp.float32)],
    )(x, w1, degt)


def _ln_relu(a, g, b):
    mu = jnp.mean(a, axis=-1, keepdims=True)
    var = jnp.mean((a - mu) ** 2, axis=-1, keepdims=True)
    h = g * (a - mu) / jnp.sqrt(var + 1e-5) + b
    return jnp.maximum(h, 0.0)


def _tcmid_body(*refs):
    s_refs = refs[:NF]
    dinv_ref, b_ref, g_ref, beta_ref, w_ref = refs[NF:NF + 5]
    xs_outs = refs[NF + 5:]
    sb = jnp.concatenate([r[...] for r in s_refs], axis=1)
    dinv = dinv_ref[...]
    a = sb * dinv + b_ref[...]
    h = _ln_relu(a, g_ref[...], beta_ref[...])
    xw = jnp.dot(h, w_ref[...],
                 preferred_element_type=jnp.float32, precision=_PREC)
    xs = xw * dinv
    for f in range(NF):
        xs_outs[f][...] = xs[:, f * 128:(f + 1) * 128]


def _tcmid(s_list, dinv, b, g, beta, w):
    return pl.pallas_call(
        _tcmid_body,
        grid=(GRID,),
        in_specs=[pl.BlockSpec((RB, 128), lambda i: (i, 0))] * NF + [
            pl.BlockSpec((RB, 1), lambda i: (i, 0)),
            pl.BlockSpec((1, D_H), lambda i: (0, 0)),
            pl.BlockSpec((1, D_H), lambda i: (0, 0)),
            pl.BlockSpec((1, D_H), lambda i: (0, 0)),
            pl.BlockSpec((D_H, D_H), lambda i: (0, 0)),
        ],
        out_specs=[pl.BlockSpec((RB, 128), lambda i: (i, 0))] * NF,
        out_shape=[jax.ShapeDtypeStruct((N, 128), jnp.float32)] * NF,
    )(*s_list, dinv, b, g, beta, w)


def _tcfin_body(*refs):
    s_refs = refs[:NF]
    dinv_ref, b_ref, g_ref, beta_ref = refs[NF:NF + 4]
    out_ref = refs[NF + 4]
    i = pl.program_id(0)
    sb = jnp.concatenate([r[...] for r in s_refs], axis=1)
    a = sb * dinv_ref[...] + b_ref[...]
    h = _ln_relu(a, g_ref[...], beta_ref[...])
    part = jnp.sum(h, axis=0, keepdims=True)

    @pl.when(i == 0)
    def _():
        out_ref[...] = part

    @pl.when(i > 0)
    def _():
        out_ref[...] = out_ref[...] + part

    @pl.when(i == GRID - 1)
    def _():
        out_ref[...] = out_ref[...] * (1.0 / N)


def _tcfin(s_list, dinv, b, g, beta):
    return pl.pallas_call(
        _tcfin_body,
        grid=(GRID,),
        in_specs=[pl.BlockSpec((RB, 128), lambda i: (i, 0))] * NF + [
            pl.BlockSpec((RB, 1), lambda i: (i, 0)),
            pl.BlockSpec((1, D_H), lambda i: (0, 0)),
            pl.BlockSpec((1, D_H), lambda i: (0, 0)),
            pl.BlockSpec((1, D_H), lambda i: (0, 0)),
        ],
        out_specs=pl.BlockSpec((1, D_H), lambda i: (0, 0)),
        out_shape=jax.ShapeDtypeStruct((1, D_H), jnp.float32),
    )(*s_list, dinv, b, g, beta)


def kernel(x, edge_index, Ws, bs, gammas, betas):
    src = edge_index[0].astype(jnp.int32)
    dst = edge_index[1].astype(jnp.int32)
    loop = jnp.arange(N, dtype=jnp.int32)
    npad = P - E2
    se = jnp.concatenate([src, loop, jnp.zeros((npad,), jnp.int32)])
    de = jnp.concatenate(
        [dst, loop, TRASH + jnp.arange(npad, dtype=jnp.int32) % NTR])

    degp = _deg_kernel(de)
    degt = degp[:, :N, 0].T

    zrows = jnp.zeros((RPT, 128), jnp.float32)

    outs = _tc1(x, Ws[0], degt)
    xs_list, dinv = list(outs[:NF]), outs[NF]
    for i in range(L - 1):
        s_list = list(_agg_kernel(se, de, zrows, *xs_list))
        xs_list = list(_tcmid(s_list, dinv, bs[i].reshape(1, D_H),
                              gammas[i].reshape(1, D_H),
                              betas[i].reshape(1, D_H), Ws[i + 1]))
    s_list = list(_agg_kernel(se, de, zrows, *xs_list))
    return _tcfin(s_list, dinv, bs[L - 1].reshape(1, D_H),
                  gammas[L - 1].reshape(1, D_H),
                  betas[L - 1].reshape(1, D_H))


# speedup vs baseline: 1.4027x; 1.1155x over previous
import functools

import jax
import jax.numpy as jnp
from jax import lax
from jax.experimental import pallas as pl
from jax.experimental.pallas import tpu as pltpu
from jax.experimental.pallas import tpu_sc as plsc

N = 10000
E = 160000
D_IN = 256
D_H = 1024
L = 4

NC = 2
NS = 16
K = 128

E2 = E + N
P = ((E2 + NC * NS * K - 1) // (NC * NS * K)) * (NC * NS * K)
PT = P // NS
NB = PT // K

NPAD = 10240
RPT = NPAD // NS
TRASH = NPAD
ACC_R = NPAD + 16
NTR = ACC_R - NPAD
NF = D_H // 128
FPC = NF // NC

RB = 1000
GRID = N // RB


_PT32 = P // (NC * NS)
_NB32 = _PT32 // K
DW = 16


def _deg_body(de_hbm, ones_hbm, zer_hbm, degp_hbm,
              acc, dst_v, ones_v, _sem):
    c = lax.axis_index("c")
    s = lax.axis_index("s")
    pltpu.sync_copy(ones_hbm, ones_v)
    pltpu.sync_copy(zer_hbm, acc.at[pl.ds(s * RPT, RPT)])
    pltpu.sync_copy(zer_hbm.at[pl.ds(0, 16)], acc.at[pl.ds(NPAD, 16)])
    plsc.subcore_barrier()

    base = (c * NS + s) * _PT32

    def batch(i, _):
        pltpu.sync_copy(de_hbm.at[pl.ds(base + i * K, K)], dst_v)
        pltpu.sync_copy(ones_v, acc.at[dst_v], add=True)
        return 0
    lax.fori_loop(0, _NB32, batch, 0)
    plsc.subcore_barrier()

    pltpu.sync_copy(acc.at[pl.ds(s * RPT, RPT)],
                    degp_hbm.at[c, pl.ds(s * RPT, RPT)])


@functools.cache
def _build_deg_kernel():
    return functools.partial(
        pl.kernel,
        out_type=jax.ShapeDtypeStruct((NC, ACC_R, DW), jnp.float32),
        mesh=plsc.VectorSubcoreMesh(core_axis_name="c", subcore_axis_name="s",
                                    num_cores=NC, num_subcores=NS),
        scratch_types=[
            pltpu.VMEM_SHARED((ACC_R, DW), jnp.float32),
            pltpu.VMEM((K,), jnp.int32),
            pltpu.VMEM((K, DW), jnp.float32),
            pltpu.SemaphoreType.DMA,
        ],
    )(_deg_body)


def _deg_kernel(de):
    ones = jnp.ones((K, DW), jnp.float32)
    zer = jnp.zeros((RPT, DW), jnp.float32)
    return _build_deg_kernel()(de, ones, zer)


def _agg_one_chunk(s, de_hbm, zrows_hbm, xs_hbm, out_hbm,
                   acc, se_v, dst_a, dst_b, rows_a, rows_b,
                   gsem_a, gsem_b, ssem_a, ssem_b, isem_a, isem_b):
    pltpu.sync_copy(zrows_hbm, acc.at[pl.ds(s * RPT, RPT)])
    plsc.subcore_barrier()

    base = s * PT

    def load_dst(i, dst_v, sem):
        pltpu.async_copy(de_hbm.at[pl.ds(base + i * K, K)], dst_v, sem)

    def start_gather(i, buf, sem):
        pltpu.async_copy(xs_hbm.at[se_v.at[pl.ds(i * K, K)]], buf, sem)

    def wait_gather(buf, sem):
        pltpu.make_async_copy(xs_hbm.at[se_v.at[pl.ds(0, K)]], buf, sem).wait()

    def wait_dst(dst_v, sem):
        pltpu.make_async_copy(de_hbm.at[pl.ds(base, K)], dst_v, sem).wait()

    def wait_scatter(buf, dst_v, sem):
        pltpu.make_async_copy(buf, acc.at[dst_v], sem).wait()

    load_dst(0, dst_a, isem_a)
    start_gather(0, rows_a, gsem_a)

    def step(g, _):
        i0 = 2 * g
        i1 = i0 + 1

        @pl.when(g > 0)
        def _():
            wait_scatter(rows_b, dst_b, ssem_b)
        load_dst(i1, dst_b, isem_b)
        start_gather(i1, rows_b, gsem_b)
        wait_gather(rows_a, gsem_a)
        wait_dst(dst_a, isem_a)
        pltpu.async_copy(rows_a, acc.at[dst_a], ssem_a, add=True)

        @pl.when(i1 + 1 < NB)
        def _():
            wait_scatter(rows_a, dst_a, ssem_a)
            load_dst(i1 + 1, dst_a, isem_a)
            start_gather(i1 + 1, rows_a, gsem_a)
        wait_gather(rows_b, gsem_b)
        wait_dst(dst_b, isem_b)
        pltpu.async_copy(rows_b, acc.at[dst_b], ssem_b, add=True)
        return 0
    lax.fori_loop(0, NB // 2, step, 0)
    wait_scatter(rows_a, dst_a, ssem_a)
    wait_scatter(rows_b, dst_b, ssem_b)
    plsc.subcore_barrier()

    @pl.when(s < NS - 1)
    def _():
        pltpu.sync_copy(acc.at[pl.ds(s * RPT, RPT)],
                        out_hbm.at[pl.ds(s * RPT, RPT)])

    @pl.when(s == NS - 1)
    def _():
        pltpu.sync_copy(acc.at[pl.ds((NS - 1) * RPT, N - (NS - 1) * RPT)],
                        out_hbm.at[pl.ds((NS - 1) * RPT, N - (NS - 1) * RPT)])

    plsc.subcore_barrier()


def _agg_body(se_hbm, de_hbm, zrows_hbm, *rest):
    xs_refs = rest[:NF]
    out_refs = rest[NF:2 * NF]
    (acc, se_v, dst_a, dst_b, rows_a, rows_b,
     gsem_a, gsem_b, ssem_a, ssem_b, isem_a, isem_b) = rest[2 * NF:]
    c = lax.axis_index("c")
    s = lax.axis_index("s")
    pltpu.sync_copy(se_hbm.at[pl.ds(s * PT, PT)], se_v)
    for j in range(FPC):
        @pl.when(c == 0)
        def _(j=j):
            _agg_one_chunk(s, de_hbm, zrows_hbm,
                           xs_refs[j], out_refs[j],
                           acc, se_v, dst_a, dst_b, rows_a, rows_b,
                           gsem_a, gsem_b, ssem_a, ssem_b, isem_a, isem_b)

        @pl.when(c == 1)
        def _(j=j):
            _agg_one_chunk(s, de_hbm, zrows_hbm,
                           xs_refs[FPC + j], out_refs[FPC + j],
                           acc, se_v, dst_a, dst_b, rows_a, rows_b,
                           gsem_a, gsem_b, ssem_a, ssem_b, isem_a, isem_b)


@functools.cache
def _build_agg_kernel():
    return functools.partial(
        pl.kernel,
        out_type=[jax.ShapeDtypeStruct((N, 128), jnp.float32)] * NF,
        mesh=plsc.VectorSubcoreMesh(core_axis_name="c", subcore_axis_name="s",
                                    num_cores=NC, num_subcores=NS),
        scratch_types=[
            pltpu.VMEM_SHARED((ACC_R, 128), jnp.float32),
            pltpu.VMEM((PT,), jnp.int32),
            pltpu.VMEM((K,), jnp.int32),
            pltpu.VMEM((K,), jnp.int32),
            pltpu.VMEM((K, 128), jnp.float32),
            pltpu.VMEM((K, 128), jnp.float32),
            pltpu.SemaphoreType.DMA,
            pltpu.SemaphoreType.DMA,
            pltpu.SemaphoreType.DMA,
            pltpu.SemaphoreType.DMA,
            pltpu.SemaphoreType.DMA,
            pltpu.SemaphoreType.DMA,
        ],
    )(_agg_body)


def _agg_kernel(se, de, zrows, *xs_list):
    return _build_agg_kernel()(se, de, zrows, *xs_list)


_PREC = lax.Precision.DEFAULT


def _tc1_body(x_ref, w_ref, degt_ref, *outs):
    xs_outs = outs[:NF]
    dinv_ref = outs[NF]
    deg = jnp.sum(degt_ref[...], axis=1, keepdims=True)
    dinv = lax.rsqrt(deg)
    xw = jnp.dot(x_ref[...], w_ref[...],
                 preferred_element_type=jnp.float32, precision=_PREC)
    xs = xw * dinv
    for f in range(NF):
        xs_outs[f][...] = xs[:, f * 128:(f + 1) * 128]
    dinv_ref[...] = dinv


def _tc1(x, w1, degt):
    return pl.pallas_call(
        _tc1_body,
        grid=(GRID,),
        in_specs=[
            pl.BlockSpec((RB, D_IN), lambda i: (i, 0)),
            pl.BlockSpec((D_IN, D_H), lambda i: (0, 0)),
            pl.BlockSpec((RB, NC), lambda i: (i, 0)),
        ],
        out_specs=[pl.BlockSpec((RB, 128), lambda i: (i, 0))] * NF
                  + [pl.BlockSpec((RB, 1), lambda i: (i, 0))],
        out_shape=[jax.ShapeDtypeStruct((N, 128), jnp.float32)] * NF
                  + [jax.ShapeDtypeStruct((N, 1), jnp.float32)],
    )(x, w1, degt)


def _ln_relu(a, g, b):
    mu = jnp.mean(a, axis=-1, keepdims=True)
    var = jnp.mean((a - mu) ** 2, axis=-1, keepdims=True)
    h = g * (a - mu) / jnp.sqrt(var + 1e-5) + b
    return jnp.maximum(h, 0.0)


def _tcmid_body(*refs):
    s_refs = refs[:NF]
    dinv_ref, b_ref, g_ref, beta_ref, w_ref = refs[NF:NF + 5]
    xs_outs = refs[NF + 5:]
    sb = jnp.concatenate([r[...] for r in s_refs], axis=1)
    dinv = dinv_ref[...]
    a = sb * dinv + b_ref[...]
    h = _ln_relu(a, g_ref[...], beta_ref[...])
    xw = jnp.dot(h, w_ref[...],
                 preferred_element_type=jnp.float32, precision=_PREC)
    xs = xw * dinv
    for f in range(NF):
        xs_outs[f][...] = xs[:, f * 128:(f + 1) * 128]


def _tcmid(s_list, dinv, b, g, beta, w):
    return pl.pallas_call(
        _tcmid_body,
        grid=(GRID,),
        in_specs=[pl.BlockSpec((RB, 128), lambda i: (i, 0))] * NF + [
            pl.BlockSpec((RB, 1), lambda i: (i, 0)),
            pl.BlockSpec((1, D_H), lambda i: (0, 0)),
            pl.BlockSpec((1, D_H), lambda i: (0, 0)),
            pl.BlockSpec((1, D_H), lambda i: (0, 0)),
            pl.BlockSpec((D_H, D_H), lambda i: (0, 0)),
        ],
        out_specs=[pl.BlockSpec((RB, 128), lambda i: (i, 0))] * NF,
        out_shape=[jax.ShapeDtypeStruct((N, 128), jnp.float32)] * NF,
    )(*s_list, dinv, b, g, beta, w)


def _tcfin_body(*refs):
    s_refs = refs[:NF]
    dinv_ref, b_ref, g_ref, beta_ref = refs[NF:NF + 4]
    out_ref = refs[NF + 4]
    i = pl.program_id(0)
    sb = jnp.concatenate([r[...] for r in s_refs], axis=1)
    a = sb * dinv_ref[...] + b_ref[...]
    h = _ln_relu(a, g_ref[...], beta_ref[...])
    part = jnp.sum(h, axis=0, keepdims=True)

    @pl.when(i == 0)
    def _():
        out_ref[...] = part

    @pl.when(i > 0)
    def _():
        out_ref[...] = out_ref[...] + part

    @pl.when(i == GRID - 1)
    def _():
        out_ref[...] = out_ref[...] * (1.0 / N)


def _tcfin(s_list, dinv, b, g, beta):
    return pl.pallas_call(
        _tcfin_body,
        grid=(GRID,),
        in_specs=[pl.BlockSpec((RB, 128), lambda i: (i, 0))] * NF + [
            pl.BlockSpec((RB, 1), lambda i: (i, 0)),
            pl.BlockSpec((1, D_H), lambda i: (0, 0)),
            pl.BlockSpec((1, D_H), lambda i: (0, 0)),
            pl.BlockSpec((1, D_H), lambda i: (0, 0)),
        ],
        out_specs=pl.BlockSpec((1, D_H), lambda i: (0, 0)),
        out_shape=jax.ShapeDtypeStruct((1, D_H), jnp.float32),
    )(*s_list, dinv, b, g, beta)


def kernel(x, edge_index, Ws, bs, gammas, betas):
    src = edge_index[0].astype(jnp.int32)
    dst = edge_index[1].astype(jnp.int32)
    loop = jnp.arange(N, dtype=jnp.int32)
    npad = P - E2
    se = jnp.concatenate([src, loop, jnp.zeros((npad,), jnp.int32)])
    de = jnp.concatenate(
        [dst, loop, TRASH + jnp.arange(npad, dtype=jnp.int32) % NTR])

    degp = _deg_kernel(de)
    degt = degp[:, :N, 0].T

    zrows = jnp.zeros((RPT, 128), jnp.float32)

    outs = _tc1(x, Ws[0], degt)
    xs_list, dinv = list(outs[:NF]), outs[NF]
    for i in range(L - 1):
        s_list = list(_agg_kernel(se, de, zrows, *xs_list))
        xs_list = list(_tcmid(s_list, dinv, bs[i].reshape(1, D_H),
                              gammas[i].reshape(1, D_H),
                              betas[i].reshape(1, D_H), Ws[i + 1]))
    s_list = list(_agg_kernel(se, de, zrows, *xs_list))
    return _tcfin(s_list, dinv, bs[L - 1].reshape(1, D_H),
                  gammas[L - 1].reshape(1, D_H),
                  betas[L - 1].reshape(1, D_H))
